# baseline (speedup 1.0000x reference)
import functools

import jax
import jax.numpy as jnp
from jax import lax
from jax.experimental import pallas as pl
from jax.experimental.pallas import tpu as pltpu
from jax.experimental.pallas import tpu_sc as plsc

N_IN = 4096
N_OUT = 4096
NNZ = 16777
BATCH = 1024

LANES = 16
NUM_CORES = 2
NUM_SUBCORES = 16
NUM_WORKERS = NUM_CORES * NUM_SUBCORES

SPACING = 11
BLOCK = LANES * SPACING
E_PAD = ((NNZ + BLOCK - 1) // BLOCK) * BLOCK

ROWS_PER_WORKER = BATCH // NUM_WORKERS
R = 8
PASSES = ROWS_PER_WORKER // R

SRC_MASK = 4095


def _sc_body(x_hbm, ew_hbm, bias_hbm, out_hbm, xbuf, outbuf, ewv, biasv):
    wid = lax.axis_index("s") * NUM_CORES + lax.axis_index("c")

    pltpu.sync_copy(ew_hbm, ewv)
    pltpu.sync_copy(bias_hbm, biasv)

    for p in range(PASSES):
        base = (wid * ROWS_PER_WORKER + p * R) * N_IN
        pltpu.sync_copy(x_hbm.at[pl.ds(base, R * N_IN)], xbuf)

        @plsc.parallel_loop(0, N_OUT, step=LANES, unroll=4)
        def _init(off):
            off = pl.multiple_of(off, LANES)
            b16 = biasv[pl.ds(off, LANES)]
            for r in range(R):
                outbuf[pl.ds(off + r * N_OUT, LANES)] = b16

        @plsc.parallel_loop(0, E_PAD, step=LANES, unroll=2)
        def _edges(off):
            off = pl.multiple_of(off, LANES)
            e16 = ewv[pl.ds(off, LANES)]
            w16 = plsc.bitcast(ewv[pl.ds(off + E_PAD, LANES)], jnp.float32)
            s16 = e16 & SRC_MASK
            d16 = lax.shift_right_logical(e16, 12)
            for r in range(R):
                vals = plsc.load_gather(xbuf, [s16 + (r * N_IN)]) * w16
                plsc.addupdate_scatter(outbuf, [d16 + (r * N_OUT)], vals)

        pltpu.sync_copy(outbuf, out_hbm.at[pl.ds(base, R * N_OUT)])


_sc_kernel = functools.partial(
    pl.kernel,
    out_type=jax.ShapeDtypeStruct((BATCH * N_OUT,), jnp.float32),
    mesh=plsc.VectorSubcoreMesh(
        core_axis_name="c", subcore_axis_name="s",
        num_cores=NUM_CORES, num_subcores=NUM_SUBCORES),
    compiler_params=pltpu.CompilerParams(needs_layout_passes=False),
    scratch_types=[
        pltpu.VMEM((R * N_IN,), jnp.float32),
        pltpu.VMEM((R * N_OUT,), jnp.float32),
        pltpu.VMEM((2 * E_PAD,), jnp.int32),
        pltpu.VMEM((N_OUT,), jnp.float32),
    ],
)(_sc_body)


def _interleave(a):
    return a.reshape(-1, LANES, SPACING).transpose(0, 2, 1).reshape(-1)


@jax.jit
def kernel(x, w_flat, bias, src_idx, dst_idx):
    pad = E_PAD - NNZ
    src = src_idx.astype(jnp.int32)
    dst = dst_idx.astype(jnp.int32)
    packed = jnp.concatenate(
        [(dst << 12) | src, jnp.zeros((pad,), jnp.int32)])
    wbits = jnp.concatenate(
        [lax.bitcast_convert_type(w_flat, jnp.int32),
         jnp.zeros((pad,), jnp.int32)])
    ew = jnp.concatenate([_interleave(packed), _interleave(wbits)])
    out = _sc_kernel(x.reshape(-1), ew, bias)
    return out.reshape(BATCH, N_OUT)


# speedup vs baseline: 1.0318x; 1.0318x over previous
import functools

import jax
import jax.numpy as jnp
from jax import lax
from jax.experimental import pallas as pl
from jax.experimental.pallas import tpu as pltpu
from jax.experimental.pallas import tpu_sc as plsc

N_IN = 4096
N_OUT = 4096
NNZ = 16777
BATCH = 1024

LANES = 16
NUM_CORES = 2
NUM_SUBCORES = 16
NUM_WORKERS = NUM_CORES * NUM_SUBCORES

SPACING = 9
BLOCK = LANES * SPACING
E_PAD = ((NNZ + BLOCK - 1) // BLOCK) * BLOCK

ROWS_PER_WORKER = BATCH // NUM_WORKERS
R = 8
PASSES = ROWS_PER_WORKER // R

SRC_MASK = 4095


def _sc_body(x_hbm, ew_hbm, bias_hbm, out_hbm, xbuf, outbuf, ewv, biasv):
    wid = lax.axis_index("s") * NUM_CORES + lax.axis_index("c")

    pltpu.sync_copy(ew_hbm, ewv)
    pltpu.sync_copy(bias_hbm, biasv)

    for p in range(PASSES):
        base = (wid * ROWS_PER_WORKER + p * R) * N_IN
        pltpu.sync_copy(x_hbm.at[pl.ds(base, R * N_IN)], xbuf)

        @plsc.parallel_loop(0, N_OUT, step=LANES, unroll=4)
        def _init(off):
            off = pl.multiple_of(off, LANES)
            b16 = biasv[pl.ds(off, LANES)]
            for r in range(R):
                outbuf[pl.ds(off + r * N_OUT, LANES)] = b16

        @plsc.parallel_loop(0, E_PAD, step=LANES, unroll=2)
        def _edges(off):
            off = pl.multiple_of(off, LANES)
            e16 = ewv[pl.ds(off, LANES)]
            w16 = plsc.bitcast(ewv[pl.ds(off + E_PAD, LANES)], jnp.float32)
            s16 = e16 & SRC_MASK
            d16 = lax.shift_right_logical(e16, 12)
            for r in range(R):
                vals = plsc.load_gather(xbuf, [s16 + (r * N_IN)]) * w16
                plsc.addupdate_scatter(outbuf, [d16 + (r * N_OUT)], vals)

        pltpu.sync_copy(outbuf, out_hbm.at[pl.ds(base, R * N_OUT)])


_sc_kernel = functools.partial(
    pl.kernel,
    out_type=jax.ShapeDtypeStruct((BATCH * N_OUT,), jnp.float32),
    mesh=plsc.VectorSubcoreMesh(
        core_axis_name="c", subcore_axis_name="s",
        num_cores=NUM_CORES, num_subcores=NUM_SUBCORES),
    compiler_params=pltpu.CompilerParams(needs_layout_passes=False),
    scratch_types=[
        pltpu.VMEM((R * N_IN,), jnp.float32),
        pltpu.VMEM((R * N_OUT,), jnp.float32),
        pltpu.VMEM((2 * E_PAD,), jnp.int32),
        pltpu.VMEM((N_OUT,), jnp.float32),
    ],
)(_sc_body)


def _interleave(a):
    return a.reshape(-1, LANES, SPACING).transpose(0, 2, 1).reshape(-1)


@jax.jit
def kernel(x, w_flat, bias, src_idx, dst_idx):
    pad = E_PAD - NNZ
    src = src_idx.astype(jnp.int32)
    dst = dst_idx.astype(jnp.int32)
    packed = jnp.concatenate(
        [(dst << 12) | src, jnp.zeros((pad,), jnp.int32)])
    wbits = jnp.concatenate(
        [lax.bitcast_convert_type(w_flat, jnp.int32),
         jnp.zeros((pad,), jnp.int32)])
    ew = jnp.concatenate([_interleave(packed), _interleave(wbits)])
    out = _sc_kernel(x.reshape(-1), ew, bias)
    return out.reshape(BATCH, N_OUT)


# speedup vs baseline: 1.0657x; 1.0328x over previous
import functools

import jax
import jax.numpy as jnp
from jax import lax
from jax.experimental import pallas as pl
from jax.experimental.pallas import tpu as pltpu
from jax.experimental.pallas import tpu_sc as plsc

N_IN = 4096
N_OUT = 4096
NNZ = 16777
BATCH = 1024

LANES = 16
NUM_CORES = 2
NUM_SUBCORES = 16
NUM_WORKERS = NUM_CORES * NUM_SUBCORES

SPACING = 7
BLOCK = LANES * SPACING
E_PAD = ((NNZ + BLOCK - 1) // BLOCK) * BLOCK

ROWS_PER_WORKER = BATCH // NUM_WORKERS
R = 8
PASSES = ROWS_PER_WORKER // R

SRC_MASK = 4095


def _sc_body(x_hbm, ew_hbm, bias_hbm, out_hbm, xbuf, outbuf, ewv, biasv):
    wid = lax.axis_index("s") * NUM_CORES + lax.axis_index("c")

    pltpu.sync_copy(ew_hbm, ewv)
    pltpu.sync_copy(bias_hbm, biasv)

    for p in range(PASSES):
        base = (wid * ROWS_PER_WORKER + p * R) * N_IN
        pltpu.sync_copy(x_hbm.at[pl.ds(base, R * N_IN)], xbuf)

        @plsc.parallel_loop(0, N_OUT, step=LANES, unroll=4)
        def _init(off):
            off = pl.multiple_of(off, LANES)
            b16 = biasv[pl.ds(off, LANES)]
            for r in range(R):
                outbuf[pl.ds(off + r * N_OUT, LANES)] = b16

        @plsc.parallel_loop(0, E_PAD, step=LANES, unroll=4)
        def _edges(off):
            off = pl.multiple_of(off, LANES)
            e16 = ewv[pl.ds(off, LANES)]
            w16 = plsc.bitcast(ewv[pl.ds(off + E_PAD, LANES)], jnp.float32)
            s16 = e16 & SRC_MASK
            d16 = lax.shift_right_logical(e16, 12)
            for r in range(R):
                vals = plsc.load_gather(xbuf, [s16 + (r * N_IN)]) * w16
                plsc.addupdate_scatter(outbuf, [d16 + (r * N_OUT)], vals)

        pltpu.sync_copy(outbuf, out_hbm.at[pl.ds(base, R * N_OUT)])


_sc_kernel = functools.partial(
    pl.kernel,
    out_type=jax.ShapeDtypeStruct((BATCH * N_OUT,), jnp.float32),
    mesh=plsc.VectorSubcoreMesh(
        core_axis_name="c", subcore_axis_name="s",
        num_cores=NUM_CORES, num_subcores=NUM_SUBCORES),
    compiler_params=pltpu.CompilerParams(needs_layout_passes=False),
    scratch_types=[
        pltpu.VMEM((R * N_IN,), jnp.float32),
        pltpu.VMEM((R * N_OUT,), jnp.float32),
        pltpu.VMEM((2 * E_PAD,), jnp.int32),
        pltpu.VMEM((N_OUT,), jnp.float32),
    ],
)(_sc_body)


def _interleave(a):
    return a.reshape(-1, LANES, SPACING).transpose(0, 2, 1).reshape(-1)


@jax.jit
def kernel(x, w_flat, bias, src_idx, dst_idx):
    pad = E_PAD - NNZ
    src = src_idx.astype(jnp.int32)
    dst = dst_idx.astype(jnp.int32)
    packed = jnp.concatenate(
        [(dst << 12) | src, jnp.zeros((pad,), jnp.int32)])
    wbits = jnp.concatenate(
        [lax.bitcast_convert_type(w_flat, jnp.int32),
         jnp.zeros((pad,), jnp.int32)])
    ew = jnp.concatenate([_interleave(packed), _interleave(wbits)])
    out = _sc_kernel(x.reshape(-1), ew, bias)
    return out.reshape(BATCH, N_OUT)


# speedup vs baseline: 1.1104x; 1.0419x over previous
import functools

import jax
import jax.numpy as jnp
from jax import lax
from jax.experimental import pallas as pl
from jax.experimental.pallas import tpu as pltpu
from jax.experimental.pallas import tpu_sc as plsc

N_IN = 4096
N_OUT = 4096
NNZ = 16777
BATCH = 1024

LANES = 16
NUM_CORES = 2
NUM_SUBCORES = 16
NUM_WORKERS = NUM_CORES * NUM_SUBCORES

SPACING = 7
BLOCK = LANES * SPACING
E_PAD = ((NNZ + BLOCK - 1) // BLOCK) * BLOCK

ROWS_PER_WORKER = BATCH // NUM_WORKERS
R = 4
PASSES = ROWS_PER_WORKER // R

SRC_MASK = 4095


def _sc_body(x_hbm, ew_hbm, bias_hbm, out_hbm,
             xbuf0, xbuf1, outbuf0, outbuf1, ewv, biasv,
             semx0, semx1, semo0, semo1, seme, semb):
    wid = lax.axis_index("s") * NUM_CORES + lax.axis_index("c")

    xb = [xbuf0, xbuf1]
    ob = [outbuf0, outbuf1]
    sx = [semx0, semx1]
    so = [semo0, semo1]

    ew_cp = pltpu.async_copy(ew_hbm, ewv, seme)
    bias_cp = pltpu.async_copy(bias_hbm, biasv, semb)

    def xbase(p):
        return (wid * ROWS_PER_WORKER + p * R) * N_IN

    x_cp = {0: pltpu.async_copy(
        x_hbm.at[pl.ds(xbase(0), R * N_IN)], xb[0], sx[0])}
    o_cp = {}

    ew_cp.wait()
    bias_cp.wait()

    for p in range(PASSES):
        b = p % 2
        x_cp[p].wait()
        if p + 1 < PASSES:
            x_cp[p + 1] = pltpu.async_copy(
                x_hbm.at[pl.ds(xbase(p + 1), R * N_IN)], xb[1 - b], sx[1 - b])
        if p >= 2:
            o_cp[p - 2].wait()

        outbuf = ob[b]
        xbuf = xb[b]

        @plsc.parallel_loop(0, N_OUT, step=LANES, unroll=4)
        def _init(off):
            off = pl.multiple_of(off, LANES)
            b16 = biasv[pl.ds(off, LANES)]
            for r in range(R):
                outbuf[pl.ds(off + r * N_OUT, LANES)] = b16

        @plsc.parallel_loop(0, E_PAD, step=LANES, unroll=2)
        def _edges(off):
            off = pl.multiple_of(off, LANES)
            e16 = ewv[pl.ds(off, LANES)]
            w16 = plsc.bitcast(ewv[pl.ds(off + E_PAD, LANES)], jnp.float32)
            s16 = e16 & SRC_MASK
            d16 = lax.shift_right_logical(e16, 12)
            for r in range(R):
                vals = plsc.load_gather(xbuf, [s16 + (r * N_IN)]) * w16
                plsc.addupdate_scatter(outbuf, [d16 + (r * N_OUT)], vals)

        o_cp[p] = pltpu.async_copy(
            outbuf, out_hbm.at[pl.ds(xbase(p), R * N_OUT)], so[b])

    o_cp[PASSES - 2].wait()
    o_cp[PASSES - 1].wait()


_sc_kernel = functools.partial(
    pl.kernel,
    out_type=jax.ShapeDtypeStruct((BATCH * N_OUT,), jnp.float32),
    mesh=plsc.VectorSubcoreMesh(
        core_axis_name="c", subcore_axis_name="s",
        num_cores=NUM_CORES, num_subcores=NUM_SUBCORES),
    compiler_params=pltpu.CompilerParams(needs_layout_passes=False),
    scratch_types=[
        pltpu.VMEM((R * N_IN,), jnp.float32),
        pltpu.VMEM((R * N_IN,), jnp.float32),
        pltpu.VMEM((R * N_OUT,), jnp.float32),
        pltpu.VMEM((R * N_OUT,), jnp.float32),
        pltpu.VMEM((2 * E_PAD,), jnp.int32),
        pltpu.VMEM((N_OUT,), jnp.float32),
        pltpu.SemaphoreType.DMA,
        pltpu.SemaphoreType.DMA,
        pltpu.SemaphoreType.DMA,
        pltpu.SemaphoreType.DMA,
        pltpu.SemaphoreType.DMA,
        pltpu.SemaphoreType.DMA,
    ],
)(_sc_body)


def _interleave(a):
    return a.reshape(-1, LANES, SPACING).transpose(0, 2, 1).reshape(-1)


@jax.jit
def kernel(x, w_flat, bias, src_idx, dst_idx):
    pad = E_PAD - NNZ
    src = src_idx.astype(jnp.int32)
    dst = dst_idx.astype(jnp.int32)
    packed = jnp.concatenate(
        [(dst << 12) | src, jnp.zeros((pad,), jnp.int32)])
    wbits = jnp.concatenate(
        [lax.bitcast_convert_type(w_flat, jnp.int32),
         jnp.zeros((pad,), jnp.int32)])
    ew = jnp.concatenate([_interleave(packed), _interleave(wbits)])
    out = _sc_kernel(x.reshape(-1), ew, bias)
    return out.reshape(BATCH, N_OUT)
